# Initial kernel scaffold; baseline (speedup 1.0000x reference)
import jax
import jax.numpy as jnp
from jax.experimental import pallas as pl


def kernel(eps_scores, ana_scores, solution_mask):
    raise NotImplementedError("write your pallas kernel here")


# speedup vs baseline: 1.6446x; 1.6446x over previous
import jax
import jax.numpy as jnp
from jax import lax
from jax.experimental import pallas as pl
from jax.experimental.pallas import tpu as pltpu

_FALSE_NEW = 0.5
_FALSE_LINK = 0.5
_WRONG_LINK = 1.0
_NEG = -1e9

_N = 4096
_BR = 512
_BC = 512
_GR = _N // _BR
_GC = _N // _BC
_DIAG_C = _BR // _BC


def _body(eps_ref, ana_ref, mask_ref, out_ref, wrong_acc, b_acc, nona_acc):
    r = pl.program_id(0)
    c = pl.program_id(1)
    last_needed = r * _DIAG_C + (_DIAG_C - 1)

    @pl.when((r == 0) & (c == 0))
    def _init_out():
        out_ref[0, 0] = 0.0

    @pl.when(c == 0)
    def _init_acc():
        wrong_acc[...] = jnp.full((_BR, _BC), _NEG, jnp.float32)
        b_acc[...] = jnp.full((_BR, _BC), _NEG, jnp.float32)
        nona_acc[...] = jnp.zeros((_BR, _BC), jnp.float32)

    @pl.when(c <= last_needed)
    def _compute():
        a = ana_ref[...]
        m = mask_ref[...] != 0
        rows = r * _BR + lax.broadcasted_iota(jnp.int32, (_BR, _BC), 0)
        cols = c * _BC + lax.broadcasted_iota(jnp.int32, (_BR, _BC), 1)
        tri = cols < rows
        diag = cols == rows
        msk = m & (tri | diag)
        eps_col = eps_ref[...]
        scores = jnp.where(diag, eps_col, jnp.where(tri, a, _NEG))
        b_acc[...] = jnp.maximum(b_acc[...], jnp.where(msk, scores, _NEG))
        wrong_acc[...] = jnp.maximum(wrong_acc[...], jnp.where(tri & ~msk, a, _NEG))
        nona_acc[...] = jnp.maximum(nona_acc[...], jnp.where(diag & msk, 1.0, 0.0))

    @pl.when(c == _GC - 1)
    def _finalize():
        wrong = jnp.max(wrong_acc[...], axis=1, keepdims=True)
        b = jnp.max(b_acc[...], axis=1, keepdims=True)
        nona = jnp.max(nona_acc[...], axis=1, keepdims=True) > 0.0
        eps_col = eps_ref[...]
        cost = jnp.where(nona, _FALSE_LINK, _WRONG_LINK)
        c1 = cost * (1.0 + wrong - b)
        c2 = jnp.where(nona, _NEG, _FALSE_NEW * (1.0 + eps_col - b))
        loss = jnp.maximum(jnp.maximum(c1, c2), 0.0)
        out_ref[0, 0] += jnp.sum(loss)


def kernel(eps_scores, ana_scores, solution_mask):
    eps2d = eps_scores.reshape(_N, 1)
    mask_i8 = solution_mask.astype(jnp.int8)

    def clamp_map(r, c):
        return (r, jnp.minimum(c, r * _DIAG_C + (_DIAG_C - 1)))

    out = pl.pallas_call(
        _body,
        grid=(_GR, _GC),
        in_specs=[
            pl.BlockSpec((_BR, 1), lambda r, c: (r, 0)),
            pl.BlockSpec((_BR, _BC), clamp_map),
            pl.BlockSpec((_BR, _BC), clamp_map),
        ],
        out_specs=pl.BlockSpec((1, 1), lambda r, c: (0, 0), memory_space=pltpu.SMEM),
        out_shape=jax.ShapeDtypeStruct((1, 1), jnp.float32),
        scratch_shapes=[
            pltpu.VMEM((_BR, _BC), jnp.float32),
            pltpu.VMEM((_BR, _BC), jnp.float32),
            pltpu.VMEM((_BR, _BC), jnp.float32),
        ],
        compiler_params=pltpu.CompilerParams(
            dimension_semantics=("arbitrary", "arbitrary"),
        ),
    )(eps2d, ana_scores, mask_i8)
    return out[0, 0]


# speedup vs baseline: 1.9926x; 1.2116x over previous
import jax
import jax.numpy as jnp
from jax import lax
from jax.experimental import pallas as pl
from jax.experimental.pallas import tpu as pltpu

_FALSE_NEW = 0.5
_FALSE_LINK = 0.5
_WRONG_LINK = 1.0
_NEG = -1e9
_NEG2 = -2e9

_N = 4096
_B = 512
_G = _N // _B


def _body(eps_ref, ana_ref, mask_ref, out_ref, wrong_acc, b_acc, nona_acc):
    r = pl.program_id(0)
    c = pl.program_id(1)

    @pl.when((r == 0) & (c == 0))
    def _init_out():
        out_ref[0, 0] = 0.0

    @pl.when(c == 0)
    def _init_acc():
        wrong_acc[...] = jnp.full((_B, _B), _NEG, jnp.float32)
        b_acc[...] = jnp.zeros((_B, _B), jnp.float32)

    @pl.when(c < r)
    def _interior():
        a = ana_ref[...]
        m = mask_ref[...].astype(jnp.float32)
        b_acc[...] += m * a
        wrong_acc[...] = jnp.maximum(wrong_acc[...], a + _NEG2 * m)

    @pl.when(c == r)
    def _diagonal():
        a = ana_ref[...]
        mi = mask_ref[...]
        m = mi.astype(jnp.float32)
        rows = lax.broadcasted_iota(jnp.int32, (_B, _B), 0)
        cols = lax.broadcasted_iota(jnp.int32, (_B, _B), 1)
        tri = cols < rows
        diag = cols == rows
        eps_col = eps_ref[...]
        scores = jnp.where(diag, eps_col, a)
        b_acc[...] += m * scores
        wrong_acc[...] = jnp.maximum(wrong_acc[...], jnp.where(tri & (mi == 0), a, _NEG))
        nona_acc[...] = jnp.where(diag & (mi != 0), 1.0, 0.0)

    @pl.when(c == _G - 1)
    def _finalize():
        wrong = jnp.max(wrong_acc[...], axis=1, keepdims=True)
        b = jnp.sum(b_acc[...], axis=1, keepdims=True)
        nona = jnp.max(nona_acc[...], axis=1, keepdims=True) > 0.0
        eps_col = eps_ref[...]
        cost = jnp.where(nona, _FALSE_LINK, _WRONG_LINK)
        c1 = cost * (1.0 + wrong - b)
        c2 = jnp.where(nona, _NEG, _FALSE_NEW * (1.0 + eps_col - b))
        loss = jnp.maximum(jnp.maximum(c1, c2), 0.0)
        out_ref[0, 0] += jnp.sum(loss)


def kernel(eps_scores, ana_scores, solution_mask):
    eps2d = eps_scores.reshape(_N, 1)
    mask_i8 = solution_mask.astype(jnp.int8)

    def clamp_map(r, c):
        return (r, jnp.minimum(c, r))

    out = pl.pallas_call(
        _body,
        grid=(_G, _G),
        in_specs=[
            pl.BlockSpec((_B, 1), lambda r, c: (r, 0)),
            pl.BlockSpec((_B, _B), clamp_map),
            pl.BlockSpec((_B, _B), clamp_map),
        ],
        out_specs=pl.BlockSpec((1, 1), lambda r, c: (0, 0), memory_space=pltpu.SMEM),
        out_shape=jax.ShapeDtypeStruct((1, 1), jnp.float32),
        scratch_shapes=[
            pltpu.VMEM((_B, _B), jnp.float32),
            pltpu.VMEM((_B, _B), jnp.float32),
            pltpu.VMEM((_B, _B), jnp.float32),
        ],
        compiler_params=pltpu.CompilerParams(
            dimension_semantics=("arbitrary", "arbitrary"),
        ),
    )(eps2d, ana_scores, mask_i8)
    return out[0, 0]
